# Initial kernel scaffold; baseline (speedup 1.0000x reference)
import functools

import jax
import jax.numpy as jnp
from jax import lax
from jax.experimental import pallas as pl
from jax.experimental.pallas import tpu as pltpu
from jax.experimental.pallas import tpu_sc as plsc

N_NODES = 10000
N_EDGES = 320000
NFEAT = 128
NEMBED = 128
NX = 64

NC = 2
NS = 16
L = 16
K = 128

ROW_BLK = 400
N_ROW_BLKS = N_NODES // ROW_BLK

CHUNKS_PER_TILE = -(-N_EDGES // (NS * K))
E_PAD = CHUNKS_PER_TILE * NS * K
NP = N_NODES + 16
ROWS_PER_TILE_ZERO = NP // NS
ROWS_PER_TILE_OUT = N_NODES // NS


def _make_spmm(fh):
    mesh = plsc.VectorSubcoreMesh(core_axis_name="c", subcore_axis_name="s")

    @functools.partial(
        pl.kernel,
        mesh=mesh,
        out_type=jax.ShapeDtypeStruct((NC * N_NODES, fh), jnp.float32),
        scratch_types=[
            pltpu.VMEM_SHARED((NP, fh), jnp.float32),
            pltpu.VMEM((K,), jnp.int32),
            pltpu.VMEM((K,), jnp.int32),
            pltpu.VMEM((K, fh), jnp.float32),
            pltpu.SemaphoreType.DMA,
        ],
    )
    def spmm(h_hbm, src_hbm, dst_hbm, zeros_hbm, out_hbm,
             acc, src_v, dst_v, rows_v, sem):
        cid = lax.axis_index("c")
        sid = lax.axis_index("s")

        z0 = sid * ROWS_PER_TILE_ZERO
        pltpu.sync_copy(zeros_hbm.at[pl.ds(z0, ROWS_PER_TILE_ZERO)],
                        acc.at[pl.ds(z0, ROWS_PER_TILE_ZERO)])
        plsc.subcore_barrier()

        row_off = cid * N_NODES

        def chunk_body(i, carry):
            base = (sid * CHUNKS_PER_TILE + i) * K
            pltpu.sync_copy(src_hbm.at[pl.ds(base, K)], src_v)
            pltpu.sync_copy(dst_hbm.at[pl.ds(base, K)], dst_v)
            for j in range(K // L):
                sl = pl.ds(j * L, L)
                src_v[sl] = src_v[sl] + row_off
            pltpu.async_copy(h_hbm.at[src_v], rows_v, sem).wait()
            pltpu.sync_copy(rows_v, acc.at[dst_v], add=True)
            return carry

        lax.fori_loop(0, CHUNKS_PER_TILE, chunk_body, 0)
        plsc.subcore_barrier()

        o0 = sid * ROWS_PER_TILE_OUT
        pltpu.sync_copy(acc.at[pl.ds(o0, ROWS_PER_TILE_OUT)],
                        out_hbm.at[pl.ds(row_off + o0, ROWS_PER_TILE_OUT)])

    return spmm


_spmm64 = _make_spmm(64)
_spmm32 = _make_spmm(32)


def _mm1_body(x_ref, w_ref, out_ref):
    out_ref[...] = jnp.dot(x_ref[...], w_ref[...],
                           preferred_element_type=jnp.float32)


def _mm1(x, w1):
    return pl.pallas_call(
        _mm1_body,
        grid=(N_ROW_BLKS, NC),
        in_specs=[
            pl.BlockSpec((ROW_BLK, NFEAT), lambda i, c: (i, 0)),
            pl.BlockSpec((NFEAT, NEMBED // NC), lambda i, c: (0, c)),
        ],
        out_specs=pl.BlockSpec((ROW_BLK, NEMBED // NC),
                               lambda i, c: (c * N_ROW_BLKS + i, 0)),
        out_shape=jax.ShapeDtypeStruct((NC * N_NODES, NEMBED // NC),
                                       jnp.float32),
    )(x, w1)


def _mm2_body(p0_ref, p1_ref, b1_ref, w2_ref, out_ref):
    h0 = jnp.maximum(p0_ref[...] + b1_ref[0, :], 0.0)
    h1 = jnp.maximum(p1_ref[...] + b1_ref[1, :], 0.0)
    out_ref[...] = (
        jnp.dot(h0, w2_ref[:64, :], preferred_element_type=jnp.float32)
        + jnp.dot(h1, w2_ref[64:, :], preferred_element_type=jnp.float32))


def _mm2(p, b1, w2):
    return pl.pallas_call(
        _mm2_body,
        grid=(N_ROW_BLKS, NC),
        in_specs=[
            pl.BlockSpec((ROW_BLK, 64), lambda i, c: (i, 0)),
            pl.BlockSpec((ROW_BLK, 64), lambda i, c: (N_ROW_BLKS + i, 0)),
            pl.BlockSpec((2, 64), lambda i, c: (0, 0)),
            pl.BlockSpec((NEMBED, NX // NC), lambda i, c: (0, c)),
        ],
        out_specs=pl.BlockSpec((ROW_BLK, NX // NC),
                               lambda i, c: (c * N_ROW_BLKS + i, 0)),
        out_shape=jax.ShapeDtypeStruct((NC * N_NODES, NX // NC), jnp.float32),
    )(p, p, b1.reshape(2, 64), w2)


def _lsm_body(q0_ref, q1_ref, b2_ref, out_ref):
    a = jnp.concatenate([q0_ref[...], q1_ref[...]], axis=1) + b2_ref[0, :]
    m = jnp.max(a, axis=1, keepdims=True)
    e = jnp.exp(a - m)
    s = jnp.sum(e, axis=1, keepdims=True)
    out_ref[...] = a - m - jnp.log(s)


def _lsm(q, b2):
    return pl.pallas_call(
        _lsm_body,
        grid=(N_ROW_BLKS,),
        in_specs=[
            pl.BlockSpec((ROW_BLK, NX // NC), lambda i: (i, 0)),
            pl.BlockSpec((ROW_BLK, NX // NC), lambda i: (N_ROW_BLKS + i, 0)),
            pl.BlockSpec((1, NX), lambda i: (0, 0)),
        ],
        out_specs=pl.BlockSpec((ROW_BLK, NX), lambda i: (i, 0)),
        out_shape=jax.ShapeDtypeStruct((N_NODES, NX), jnp.float32),
    )(q, q, b2.reshape(1, NX))


def kernel(x, edge_index, W1, b1, W2, b2):
    src = edge_index[0].astype(jnp.int32)
    dst = edge_index[1].astype(jnp.int32)
    pad = E_PAD - N_EDGES
    src = jnp.concatenate([src, jnp.zeros((pad,), jnp.int32)])
    dst = jnp.concatenate([dst, jnp.full((pad,), N_NODES, jnp.int32)])
    zeros64 = jnp.zeros((NP, 64), jnp.float32)
    zeros32 = jnp.zeros((NP, 32), jnp.float32)

    h = _mm1(x, W1)
    p = _spmm64(h, src, dst, zeros64)
    h2 = _mm2(p, b1, W2)
    q = _spmm32(h2, src, dst, zeros32)
    return _lsm(q, b2)


# speedup vs baseline: 3.8569x; 3.8569x over previous
import functools

import jax
import jax.numpy as jnp
from jax import lax
from jax.experimental import pallas as pl
from jax.experimental.pallas import tpu as pltpu
from jax.experimental.pallas import tpu_sc as plsc

N_NODES = 10000
N_EDGES = 320000
NFEAT = 128
NEMBED = 128
NX = 64

NC = 2
NS = 16
K = 128

ROW_BLK = 400
N_ROW_BLKS = N_NODES // ROW_BLK

CHUNKS_PER_TILE = -(-N_EDGES // (NC * NS * K))
E_PAD = CHUNKS_PER_TILE * NC * NS * K
NP = N_NODES + 112
ROWS_PER_TILE_ZERO = NP // NS
ROWS_PER_TILE_OUT = 624
OUT_TAIL_BASE = NS * ROWS_PER_TILE_OUT
OUT_TAIL = N_NODES - OUT_TAIL_BASE

_mesh = plsc.VectorSubcoreMesh(core_axis_name="c", subcore_axis_name="s")


@functools.partial(
    pl.kernel,
    mesh=_mesh,
    out_type=jax.ShapeDtypeStruct((NC * N_NODES, NFEAT), jnp.float32),
    scratch_types=[
        pltpu.VMEM_SHARED((NP, NFEAT), jnp.float32),
        pltpu.VMEM((K,), jnp.int32),
        pltpu.VMEM((K,), jnp.int32),
        pltpu.VMEM((K, NFEAT), jnp.float32),
        pltpu.SemaphoreType.DMA,
    ],
)
def _spmm(h_hbm, src_hbm, dst_hbm, zeros_hbm, out_hbm,
          acc, src_v, dst_v, rows_v, sem):
    cid = lax.axis_index("c")
    sid = lax.axis_index("s")

    z0 = sid * ROWS_PER_TILE_ZERO
    pltpu.sync_copy(zeros_hbm.at[pl.ds(z0, ROWS_PER_TILE_ZERO)],
                    acc.at[pl.ds(z0, ROWS_PER_TILE_ZERO)])
    plsc.subcore_barrier()

    tile_chunk0 = (cid * NS + sid) * CHUNKS_PER_TILE

    def chunk_body(i, carry):
        base = (tile_chunk0 + i) * K
        pltpu.sync_copy(src_hbm.at[pl.ds(base, K)], src_v)
        pltpu.sync_copy(dst_hbm.at[pl.ds(base, K)], dst_v)
        pltpu.async_copy(h_hbm.at[src_v], rows_v, sem).wait()
        pltpu.sync_copy(rows_v, acc.at[dst_v], add=True)
        return carry

    lax.fori_loop(0, CHUNKS_PER_TILE, chunk_body, 0)
    plsc.subcore_barrier()

    row_off = cid * N_NODES
    o0 = sid * ROWS_PER_TILE_OUT
    pltpu.sync_copy(acc.at[pl.ds(o0, ROWS_PER_TILE_OUT)],
                    out_hbm.at[pl.ds(row_off + o0, ROWS_PER_TILE_OUT)])

    @pl.when(sid == 0)
    def _copy_tail():
        pltpu.sync_copy(acc.at[pl.ds(OUT_TAIL_BASE, OUT_TAIL)],
                        out_hbm.at[pl.ds(row_off + OUT_TAIL_BASE, OUT_TAIL)])


def _mm1_body(x_ref, w_ref, out_ref):
    out_ref[...] = jnp.dot(x_ref[...], w_ref[...],
                           preferred_element_type=jnp.float32)


def _mm1(x, w1):
    return pl.pallas_call(
        _mm1_body,
        grid=(N_ROW_BLKS,),
        in_specs=[
            pl.BlockSpec((ROW_BLK, NFEAT), lambda i: (i, 0)),
            pl.BlockSpec((NFEAT, NEMBED), lambda i: (0, 0)),
        ],
        out_specs=pl.BlockSpec((ROW_BLK, NEMBED), lambda i: (i, 0)),
        out_shape=jax.ShapeDtypeStruct((N_NODES, NEMBED), jnp.float32),
    )(x, w1)


def _relu_body(p0_ref, p1_ref, b1_ref, out_ref):
    out_ref[...] = jnp.maximum(p0_ref[...] + p1_ref[...] + b1_ref[0, :], 0.0)


def _relu_combine(p, b1):
    return pl.pallas_call(
        _relu_body,
        grid=(N_ROW_BLKS,),
        in_specs=[
            pl.BlockSpec((ROW_BLK, NEMBED), lambda i: (i, 0)),
            pl.BlockSpec((ROW_BLK, NEMBED), lambda i: (N_ROW_BLKS + i, 0)),
            pl.BlockSpec((1, NEMBED), lambda i: (0, 0)),
        ],
        out_specs=pl.BlockSpec((ROW_BLK, NEMBED), lambda i: (i, 0)),
        out_shape=jax.ShapeDtypeStruct((N_NODES, NEMBED), jnp.float32),
    )(p, p, b1.reshape(1, NEMBED))


def _lsm_body(q0_ref, q1_ref, w2_ref, b2_ref, out_ref):
    s = q0_ref[...] + q1_ref[...]
    a = jnp.dot(s, w2_ref[...], preferred_element_type=jnp.float32) + b2_ref[0, :]
    m = jnp.max(a, axis=1, keepdims=True)
    e = jnp.exp(a - m)
    out_ref[...] = a - m - jnp.log(jnp.sum(e, axis=1, keepdims=True))


def _lsm(q, w2, b2):
    return pl.pallas_call(
        _lsm_body,
        grid=(N_ROW_BLKS,),
        in_specs=[
            pl.BlockSpec((ROW_BLK, NEMBED), lambda i: (i, 0)),
            pl.BlockSpec((ROW_BLK, NEMBED), lambda i: (N_ROW_BLKS + i, 0)),
            pl.BlockSpec((NEMBED, NX), lambda i: (0, 0)),
            pl.BlockSpec((1, NX), lambda i: (0, 0)),
        ],
        out_specs=pl.BlockSpec((ROW_BLK, NX), lambda i: (i, 0)),
        out_shape=jax.ShapeDtypeStruct((N_NODES, NX), jnp.float32),
    )(q, q, w2, b2.reshape(1, NX))


def kernel(x, edge_index, W1, b1, W2, b2):
    src = edge_index[0].astype(jnp.int32)
    dst = edge_index[1].astype(jnp.int32)
    pad = E_PAD - N_EDGES
    src = jnp.concatenate([src, jnp.zeros((pad,), jnp.int32)])
    dst = jnp.concatenate([dst, jnp.full((pad,), N_NODES, jnp.int32)])
    zeros = jnp.zeros((NP, NFEAT), jnp.float32)

    h = _mm1(x, W1)
    p = _spmm(h, src, dst, zeros)
    h1 = _relu_combine(p, b1)
    q = _spmm(h1, src, dst, zeros)
    return _lsm(q, W2, b2)
